# baseline (speedup 1.0000x reference)
import jax
import jax.numpy as jnp
from jax import lax
from jax.experimental import pallas as pl
from jax.experimental.pallas import tpu as pltpu
from jax.experimental.pallas import tpu_sc as plsc

_N_NODES = 10000
_N_EDGES = 320000
_N_PAIRS = _N_EDGES // 2
_D = 128
_NT = 512
_ET = 64
_EPS = 1e-5
_NODE_PAD = 10240

_NC = 2
_NS = 16
_NW = _NC * _NS
_CP = 200
_GP = 136
_PAIR_CHUNKS = _N_PAIRS // (_NW * _CP)


def _stats_body(nt_row_ref, et_ref, ete_ref, eto_ref, ntab_ref, etab_ref,
                gn_ref, bn_ref, ge_ref, be_ref,
                x_ref, oute_ref, ptab_ref, pidx_ref):
    def norm(tab, cnt, n, g, b):
        mean = jnp.sum(tab * cnt, axis=0, keepdims=True) / n
        msq = jnp.sum(tab * tab * cnt, axis=0, keepdims=True) / n
        var = msq - mean * mean
        scale = g * lax.rsqrt(var + _EPS)
        shift = b - mean * scale
        return tab * scale + shift

    bins_n = lax.broadcasted_iota(jnp.int32, (_NT, 1), 0)
    n_chunks = []
    cn = jnp.zeros((_NT, 1), jnp.float32)
    for c in range(_NODE_PAD // 1280):
        ids = nt_row_ref[:, pl.ds(c * 1280, 1280)]
        oh = (ids == bins_n).astype(jnp.float32)
        n_chunks.append(oh)
        cn = cn + jnp.sum(oh, axis=1, keepdims=True)
    norm_nt = norm(ntab_ref[...], cn, float(_N_NODES), gn_ref[...], bn_ref[...])
    for c, oh in enumerate(n_chunks):
        x_ref[pl.ds(c * 1280, 1280), :] = lax.dot_general(
            oh, norm_nt, (((0,), (0,)), ((), ())),
            preferred_element_type=jnp.float32)

    bins_e = lax.broadcasted_iota(jnp.int32, (_ET, 1, 1), 0)
    acc = jnp.zeros((_ET, _D), jnp.float32)
    for c in range(10):
        chunk = et_ref[pl.ds(c * 256, 256), :]
        acc = acc + jnp.sum((chunk[None, :, :] == bins_e).astype(jnp.float32),
                            axis=1)
    ce = jnp.sum(acc, axis=1, keepdims=True)
    norm_et = norm(etab_ref[...], ce, float(_N_EDGES), ge_ref[...], be_ref[...])
    oute_ref[...] = norm_et

    left = jnp.broadcast_to(norm_et[:, None, :], (_ET, _ET, _D))
    right = jnp.broadcast_to(norm_et[None, :, :], (_ET, _ET, _D))
    ptab_ref[:, pl.ds(0, _D)] = left.reshape(_ET * _ET, _D)
    ptab_ref[:, pl.ds(_D, _D)] = right.reshape(_ET * _ET, _D)

    pidx_ref[...] = ete_ref[...] * _ET + eto_ref[...]


def _edge_body(pidx_hbm, etab_hbm, ptab_hbm, oute_hbm,
               pidx_v, tab_v, buf_a, buf_b,
               sem_ga, sem_gb, sem_sa, sem_sb):
    wid = lax.axis_index("s") * _NC + lax.axis_index("c")
    bufs = (buf_a, buf_b)
    sem_g = (sem_ga, sem_gb)
    sem_s = (sem_sa, sem_sb)

    pltpu.sync_copy(etab_hbm, tab_v)
    base = wid * (_PAIR_CHUNKS * _CP)
    pltpu.sync_copy(pidx_hbm.at[pl.ds(base, _PAIR_CHUNKS * _CP)], pidx_v)

    def _build_pairs(j, buf):
        def _group(g, carry):
            p0 = _GP + g * 16
            pvec = pidx_v[pl.ds(j * _CP + p0, 16)]
            for l in range(16):
                pid = pvec[l]
                a = lax.shift_right_logical(pid, 6)
                b = lax.bitwise_and(pid, _ET - 1)
                for k in range(_D // 16):
                    buf[p0 + l, pl.ds(k * 16, 16)] = tab_v[a, pl.ds(k * 16, 16)]
                for k in range(_D // 16):
                    buf[p0 + l, pl.ds(_D + k * 16, 16)] = tab_v[b, pl.ds(k * 16, 16)]
            return carry
        lax.fori_loop(0, (_CP - _GP) // 16, _group, 0)

    def _step(j, carry):
        def _phase(x):
            @pl.when(j > 1)
            def _():
                pltpu.make_async_copy(oute_hbm.at[pl.ds(0, _CP)], bufs[x],
                                      sem_s[x]).wait()
            pltpu.async_copy(ptab_hbm.at[pidx_v.at[pl.ds(j * _CP, _GP)]],
                             bufs[x].at[pl.ds(0, _GP)], sem_g[x])
            _build_pairs(j, bufs[x])
            pltpu.make_async_copy(oute_hbm.at[pl.ds(0, _GP)],
                                  bufs[x].at[pl.ds(0, _GP)], sem_g[x]).wait()
            pltpu.async_copy(bufs[x], oute_hbm.at[pl.ds(base + j * _CP, _CP)],
                             sem_s[x])

        @pl.when(lax.rem(j, 2) == 0)
        def _even():
            _phase(0)

        @pl.when(lax.rem(j, 2) == 1)
        def _odd():
            _phase(1)

        return carry

    lax.fori_loop(0, _PAIR_CHUNKS, _step, 0)
    pltpu.make_async_copy(oute_hbm.at[pl.ds(0, _CP)], buf_b, sem_sb).wait()
    pltpu.make_async_copy(oute_hbm.at[pl.ds(0, _CP)], buf_a, sem_sa).wait()


def kernel(node_type, edge_type, node_table, edge_table,
           node_gamma, node_beta, edge_gamma, edge_beta):
    nt = node_type.astype(jnp.int32)
    et = edge_type.astype(jnp.int32)

    nt_p = jnp.concatenate([nt, jnp.full((_NODE_PAD - _N_NODES,), _NT,
                                         jnp.int32)])
    et_p = jnp.concatenate([et, jnp.full((327680 - _N_EDGES,), _ET, jnp.int32)])
    et2 = et.reshape(_N_PAIRS, 2)

    x_pad, norm_et, pair_tab, pidx = pl.pallas_call(
        _stats_body,
        out_shape=(jax.ShapeDtypeStruct((_NODE_PAD, _D), jnp.float32),
                   jax.ShapeDtypeStruct((_ET, _D), jnp.float32),
                   jax.ShapeDtypeStruct((_ET * _ET, 2 * _D), jnp.float32),
                   jax.ShapeDtypeStruct((_N_PAIRS // _D, _D), jnp.int32)),
    )(nt_p.reshape(1, _NODE_PAD), et_p.reshape(2560, 128),
      et2[:, 0].reshape(_N_PAIRS // _D, _D),
      et2[:, 1].reshape(_N_PAIRS // _D, _D),
      node_table, edge_table,
      node_gamma.reshape(1, _D), node_beta.reshape(1, _D),
      edge_gamma.reshape(1, _D), edge_beta.reshape(1, _D))

    mesh = plsc.VectorSubcoreMesh(core_axis_name="c", subcore_axis_name="s",
                                  num_cores=_NC, num_subcores=_NS)
    edge_pairs = pl.kernel(
        _edge_body,
        out_type=jax.ShapeDtypeStruct((_N_PAIRS, 2 * _D), jnp.float32),
        mesh=mesh,
        scratch_types=[pltpu.VMEM((_PAIR_CHUNKS * _CP,), jnp.int32),
                       pltpu.VMEM((_ET, _D), jnp.float32),
                       pltpu.VMEM((_CP, 2 * _D), jnp.float32),
                       pltpu.VMEM((_CP, 2 * _D), jnp.float32),
                       pltpu.SemaphoreType.DMA,
                       pltpu.SemaphoreType.DMA,
                       pltpu.SemaphoreType.DMA,
                       pltpu.SemaphoreType.DMA],
    )(pidx.reshape(_N_PAIRS), norm_et, pair_tab)

    return (x_pad[:_N_NODES], edge_pairs.reshape(_N_EDGES, _D))
